# Initial kernel scaffold; baseline (speedup 1.0000x reference)
import jax
import jax.numpy as jnp
from jax.experimental import pallas as pl


def kernel(input, pos_table):
    raise NotImplementedError("write your pallas kernel here")


# speedup vs baseline: 1.6954x; 1.6954x over previous
import jax
import jax.numpy as jnp
from jax.experimental import pallas as pl

S, B, E = 8192, 4, 1024
BS = 512


def _add_body(in_ref, tab_ref, out_ref):
    out_ref[...] = in_ref[...] + tab_ref[...][:, None, :]


def kernel(input, pos_table):
    return pl.pallas_call(
        _add_body,
        grid=(S // BS,),
        in_specs=[
            pl.BlockSpec((BS, B, E), lambda i: (i, 0, 0)),
            pl.BlockSpec((BS, E), lambda i: (i, 0)),
        ],
        out_specs=pl.BlockSpec((BS, B, E), lambda i: (i, 0, 0)),
        out_shape=jax.ShapeDtypeStruct((S, B, E), jnp.float32),
    )(input, pos_table)
